# Initial kernel scaffold; baseline (speedup 1.0000x reference)
import jax
import jax.numpy as jnp
from jax.experimental import pallas as pl


def kernel(x, tables):
    raise NotImplementedError("write your pallas kernel here")


# speedup vs baseline: 10.4711x; 10.4711x over previous
import functools

import jax
import jax.numpy as jnp
from jax import lax
from jax.experimental import pallas as pl
from jax.experimental.pallas import tpu as pltpu
from jax.experimental.pallas import tpu_sc as plsc

_N_FIELDS = 26
_VOCAB = 100000
_DIM = 32
_LANES = 16
_PERIOD = 208

_NC = 2
_NS = 16
_NW = _NC * _NS

_CH = 32


def _sc_embed(idx_flat, table_flat, off, *, rows_total):
    per_w = rows_total // _NW
    n_chunks = per_w // _CH
    mesh = plsc.VectorSubcoreMesh(core_axis_name="c", subcore_axis_name="s")

    @functools.partial(
        pl.kernel,
        mesh=mesh,
        compiler_params=pltpu.CompilerParams(use_tc_tiling_on_sc=False),
        out_type=jax.ShapeDtypeStruct((rows_total, _DIM), jnp.float32),
        scratch_types=[
            pltpu.VMEM((_CH * _N_FIELDS,), jnp.int32),
            pltpu.VMEM((_CH * _N_FIELDS, _DIM), jnp.float32),
            pltpu.VMEM((_CH, _DIM), jnp.float32),
            pltpu.VMEM((_PERIOD,), jnp.int32),
            pltpu.SemaphoreType.DMA,
        ],
    )
    def k(idx_hbm, tab_hbm, off_hbm, out_hbm, idx_v, rows_v, out_v, off_v, sem):
        wid = lax.axis_index("s") * _NC + lax.axis_index("c")
        base = wid * per_w
        pltpu.sync_copy(off_hbm, off_v)

        def chunk_body(c, carry):
            row0 = base + c * _CH
            pltpu.sync_copy(idx_hbm.at[pl.ds(row0 * _N_FIELDS, _CH * _N_FIELDS)],
                            idx_v)

            def off_body(j, carry2):
                g = idx_v[pl.ds(j * _LANES, _LANES)]
                o = off_v[pl.ds((j % 13) * _LANES, _LANES)]
                idx_v[pl.ds(j * _LANES, _LANES)] = g + o
                return carry2

            lax.fori_loop(0, (_CH * _N_FIELDS) // _LANES, off_body, 0)

            pltpu.async_copy(tab_hbm.at[idx_v], rows_v, sem).wait()

            def row_body(r, carry2):
                b = r * _N_FIELDS
                acc0 = rows_v[b, pl.ds(0, _LANES)]
                acc1 = rows_v[b, pl.ds(_LANES, _LANES)]
                for i in range(1, _N_FIELDS):
                    acc0 = acc0 + rows_v[b + i, pl.ds(0, _LANES)]
                    acc1 = acc1 + rows_v[b + i, pl.ds(_LANES, _LANES)]
                out_v[r, pl.ds(0, _LANES)] = acc0
                out_v[r, pl.ds(_LANES, _LANES)] = acc1
                return carry2

            lax.fori_loop(0, _CH, row_body, 0)
            pltpu.sync_copy(out_v, out_hbm.at[pl.ds(row0, _CH)])
            return carry

        lax.fori_loop(0, n_chunks, chunk_body, 0)

    return k(idx_flat, table_flat, off)


def kernel(x, tables):
    b, h, w, n = x.shape
    rows_total = b * h * w
    idx_flat = x.reshape(-1).astype(jnp.int32)
    table_flat = tables.reshape(n * _VOCAB, _DIM)
    off = ((jnp.arange(_PERIOD, dtype=jnp.int32) % _N_FIELDS) * _VOCAB)
    out = _sc_embed(idx_flat, table_flat, off, rows_total=rows_total)
    return out.reshape(b, h, w, _DIM)


# speedup vs baseline: 11.8985x; 1.1363x over previous
import functools

import jax
import jax.numpy as jnp
from jax import lax
from jax.experimental import pallas as pl
from jax.experimental.pallas import tpu as pltpu
from jax.experimental.pallas import tpu_sc as plsc

_N_FIELDS = 26
_VOCAB = 100000
_DIM = 32
_LANES = 16

_NC = 2
_NS = 16
_NW = _NC * _NS

_CH = 128


def _sc_embed(xt, tables, *, rows_total):
    per_w = rows_total // _NW
    n_chunks = per_w // _CH
    mesh = plsc.VectorSubcoreMesh(core_axis_name="c", subcore_axis_name="s")

    @functools.partial(
        pl.kernel,
        mesh=mesh,
        compiler_params=pltpu.CompilerParams(use_tc_tiling_on_sc=False),
        out_type=jax.ShapeDtypeStruct((rows_total, _DIM), jnp.float32),
        scratch_types=[
            pltpu.VMEM((_N_FIELDS, _CH), jnp.int32),
            pltpu.VMEM((_N_FIELDS * _CH, _DIM), jnp.float32),
            pltpu.VMEM((_CH, _DIM), jnp.float32),
            pltpu.SemaphoreType.DMA,
        ],
    )
    def k(xt_hbm, tab_hbm, out_hbm, idx_v, rows_v, out_v, sem):
        wid = lax.axis_index("s") * _NC + lax.axis_index("c")
        base = wid * per_w

        def chunk_body(c, carry):
            row0 = base + c * _CH
            pltpu.sync_copy(xt_hbm.at[:, pl.ds(row0, _CH)], idx_v)
            copies = [
                pltpu.async_copy(tab_hbm.at[i].at[idx_v.at[i]],
                                 rows_v.at[pl.ds(i * _CH, _CH)], sem)
                for i in range(_N_FIELDS)
            ]
            for cp in copies:
                cp.wait()

            def row_body(r, carry2):
                acc0 = rows_v[r, pl.ds(0, _LANES)]
                acc1 = rows_v[r, pl.ds(_LANES, _LANES)]
                for i in range(1, _N_FIELDS):
                    acc0 = acc0 + rows_v[i * _CH + r, pl.ds(0, _LANES)]
                    acc1 = acc1 + rows_v[i * _CH + r, pl.ds(_LANES, _LANES)]
                out_v[r, pl.ds(0, _LANES)] = acc0
                out_v[r, pl.ds(_LANES, _LANES)] = acc1
                return carry2

            lax.fori_loop(0, _CH, row_body, 0)
            pltpu.sync_copy(out_v, out_hbm.at[pl.ds(row0, _CH)])
            return carry

        lax.fori_loop(0, n_chunks, chunk_body, 0)

    return k(xt, tables)


def kernel(x, tables):
    b, h, w, n = x.shape
    rows_total = b * h * w
    xt = x.reshape(rows_total, n).T.astype(jnp.int32)
    out = _sc_embed(xt, tables, rows_total=rows_total)
    return out.reshape(b, h, w, _DIM)
